# baseline (speedup 1.0000x reference)
import jax
import jax.numpy as jnp
from jax import lax
from jax.experimental import pallas as pl
from jax.experimental.pallas import tpu as pltpu
from jax.experimental.pallas import tpu_sc as plsc

_B, _N, _L = 16, 1024, 8
_LANES = 16
_NC, _NS = 2, 16
_GPT = _L // _NC
_CH = 256


def _tc_count_body(vals_ref, labs_ref, cnt_ref):
    vals = vals_ref[0, 0, :]
    labs = labs_ref[0, 0, :]
    counts = []
    for c in range(_N // _CH):
        vi = vals[c * _CH:(c + 1) * _CH][:, None]
        li = labs[c * _CH:(c + 1) * _CH][:, None]
        eq = (vi == vals[None, :]) & (li == labs[None, :])
        counts.append(jnp.sum(eq.astype(jnp.float32), axis=1))
    cnt_ref[0, 0, :] = jnp.concatenate(counts)


def _gather16(x, idx):
    return x.at[idx].get(mode="promise_in_bounds")


def _sc_stats_body(vals_hbm, labs_hbm, cnt_hbm, out_hbm,
                   vals_v, labs_v, cnt_v, out_v):
    c = lax.axis_index("c")
    s = lax.axis_index("s")
    pltpu.sync_copy(vals_hbm.at[s], vals_v)
    pltpu.sync_copy(labs_hbm.at[s], labs_v)
    pltpu.sync_copy(cnt_hbm.at[s], cnt_v)

    iota = lax.broadcasted_iota(jnp.int32, (_LANES,), 0)
    total = jnp.zeros((_LANES,), jnp.float32)

    for k in range(_GPT):
        gf = jnp.full((_LANES,), c * _GPT + k, jnp.int32).astype(jnp.float32)

        def pass1(jv, carry, gf=gf):
            gmx, gsm, gct = carry
            vv = vals_v[pl.ds(jv * _LANES, _LANES)]
            lv = labs_v[pl.ds(jv * _LANES, _LANES)]
            cv = cnt_v[pl.ds(jv * _LANES, _LANES)]
            m = lv == gf
            gmx = jnp.maximum(gmx, jnp.where(m, cv, jnp.float32(-1.0)))
            gsm = gsm + jnp.where(m, vv, jnp.float32(0.0))
            gct = gct + jnp.where(m, jnp.float32(1.0), jnp.float32(0.0))
            return gmx, gsm, gct

        gmx, gsm, gct = lax.fori_loop(
            0, _N // _LANES, pass1,
            (jnp.full((_LANES,), -1.0, jnp.float32),
             jnp.zeros((_LANES,), jnp.float32),
             jnp.zeros((_LANES,), jnp.float32)))
        for d in (1, 2, 4, 8):
            gmx = jnp.maximum(gmx, _gather16(gmx, iota ^ d))
            gsm = gsm + _gather16(gsm, iota ^ d)
            gct = gct + _gather16(gct, iota ^ d)

        def pass2(jv, acc, gf=gf, gmx=gmx):
            vv = vals_v[pl.ds(jv * _LANES, _LANES)]
            lv = labs_v[pl.ds(jv * _LANES, _LANES)]
            cv = cnt_v[pl.ds(jv * _LANES, _LANES)]
            m = (lv == gf) & (cv == gmx)
            return jnp.minimum(acc, jnp.where(m, vv, jnp.float32(jnp.inf)))

        mode = lax.fori_loop(0, _N // _LANES, pass2,
                             jnp.full((_LANES,), jnp.inf, jnp.float32))
        for d in (1, 2, 4, 8):
            mode = jnp.minimum(mode, _gather16(mode, iota ^ d))

        total = total + jnp.where(gct > jnp.float32(0.0),
                                  gsm - gct * mode, jnp.float32(0.0))

    out_v[...] = total
    pltpu.sync_copy(out_v, out_hbm.at[s * _NC + c])


def kernel(inputs, targets):
    cnt = pl.pallas_call(
        _tc_count_body,
        grid=(_B,),
        in_specs=[
            pl.BlockSpec((1, 1, _N), lambda b: (b, 0, 0)),
            pl.BlockSpec((1, 1, _N), lambda b: (b, 0, 0)),
        ],
        out_specs=pl.BlockSpec((1, 1, _N), lambda b: (b, 0, 0)),
        out_shape=jax.ShapeDtypeStruct((_B, 1, _N), jnp.float32),
    )(inputs.reshape(_B, 1, _N), targets.reshape(_B, 1, _N))

    mesh = plsc.VectorSubcoreMesh(
        core_axis_name="c", subcore_axis_name="s",
        num_cores=_NC, num_subcores=_NS)
    partials = pl.kernel(
        _sc_stats_body,
        out_type=jax.ShapeDtypeStruct((_NC * _NS, _LANES), jnp.float32),
        mesh=mesh,
        scratch_types=[
            pltpu.VMEM((_N,), jnp.float32),
            pltpu.VMEM((_N,), jnp.float32),
            pltpu.VMEM((_N,), jnp.float32),
            pltpu.VMEM((_LANES,), jnp.float32),
        ],
    )(inputs, targets, cnt.reshape(_B, _N))
    return jnp.sum(partials[:, 0]) / jnp.float32(_B * _N)


# speedup vs baseline: 1.6759x; 1.6759x over previous
import jax
import jax.numpy as jnp
from jax import lax
from jax.experimental import pallas as pl
from jax.experimental.pallas import tpu as pltpu

_B, _N, _L = 16, 1024, 8
_CH = 256


def _row_body(vals_ref, labs_ref, out_ref):
    b = pl.program_id(0)
    vals = vals_ref[0, 0, :]
    labs = labs_ref[0, 0, :]

    counts = []
    for c in range(_N // _CH):
        vi = vals[c * _CH:(c + 1) * _CH][:, None]
        li = labs[c * _CH:(c + 1) * _CH][:, None]
        eq = (vi == vals[None, :]) & (li == labs[None, :])
        counts.append(jnp.sum(eq.astype(jnp.float32), axis=1))
    count = jnp.concatenate(counts)

    gids = lax.broadcasted_iota(jnp.int32, (_L, 1), 0).astype(jnp.float32)
    m = labs[None, :] == gids
    gsize = jnp.sum(jnp.where(m, 1.0, 0.0), axis=1)
    gsum = jnp.sum(jnp.where(m, vals[None, :], 0.0), axis=1)
    gmax = jnp.max(jnp.where(m, count[None, :], -1.0), axis=1)
    cand = m & (count[None, :] == gmax[:, None])
    mode = jnp.min(jnp.where(cand, vals[None, :], jnp.inf), axis=1)
    contrib = jnp.where(gsize > 0, gsum - gsize * mode, 0.0)
    total = jnp.sum(contrib)

    @pl.when(b == 0)
    def _():
        out_ref[0, 0] = jnp.float32(0.0)

    out_ref[0, 0] += total / jnp.float32(_B * _N)


def kernel(inputs, targets):
    out = pl.pallas_call(
        _row_body,
        grid=(_B,),
        in_specs=[
            pl.BlockSpec((1, 1, _N), lambda b: (b, 0, 0)),
            pl.BlockSpec((1, 1, _N), lambda b: (b, 0, 0)),
        ],
        out_specs=pl.BlockSpec((1, 1), lambda b: (0, 0), memory_space=pltpu.SMEM),
        out_shape=jax.ShapeDtypeStruct((1, 1), jnp.float32),
    )(inputs.reshape(_B, 1, _N), targets.reshape(_B, 1, _N))
    return out[0, 0]


# speedup vs baseline: 2.3524x; 1.4037x over previous
import jax
import jax.numpy as jnp
from jax import lax
from jax.experimental import pallas as pl
from jax.experimental.pallas import tpu as pltpu

_B, _N, _L = 16, 1024, 8
_CH = 256


def _row_body(vals_ref, labs_ref, out_ref):
    b = pl.program_id(0)
    vals = vals_ref[0, 0, :]
    labs = labs_ref[0, 0, :]

    count = jnp.zeros((_N,), jnp.float32)
    for c in range(_N // _CH):
        vi = vals[c * _CH:(c + 1) * _CH][:, None]
        li = labs[c * _CH:(c + 1) * _CH][:, None]
        eq = (vi == vals[None, :]) & (li == labs[None, :])
        count = count + jnp.sum(eq.astype(jnp.float32), axis=0)

    gids = lax.broadcasted_iota(jnp.int32, (_L, 1), 0).astype(jnp.float32)
    m = labs[None, :] == gids
    gsize = jnp.sum(jnp.where(m, 1.0, 0.0), axis=1)
    gsum = jnp.sum(jnp.where(m, vals[None, :], 0.0), axis=1)
    gmax = jnp.max(jnp.where(m, count[None, :], -1.0), axis=1)
    cand = m & (count[None, :] == gmax[:, None])
    mode = jnp.min(jnp.where(cand, vals[None, :], jnp.inf), axis=1)
    contrib = jnp.where(gsize > 0, gsum - gsize * mode, 0.0)
    total = jnp.sum(contrib)

    @pl.when(b == 0)
    def _():
        out_ref[0, 0] = jnp.float32(0.0)

    out_ref[0, 0] += total / jnp.float32(_B * _N)


def kernel(inputs, targets):
    out = pl.pallas_call(
        _row_body,
        grid=(_B,),
        in_specs=[
            pl.BlockSpec((1, 1, _N), lambda b: (b, 0, 0)),
            pl.BlockSpec((1, 1, _N), lambda b: (b, 0, 0)),
        ],
        out_specs=pl.BlockSpec((1, 1), lambda b: (0, 0), memory_space=pltpu.SMEM),
        out_shape=jax.ShapeDtypeStruct((1, 1), jnp.float32),
    )(inputs.reshape(_B, 1, _N), targets.reshape(_B, 1, _N))
    return out[0, 0]


# speedup vs baseline: 2.3920x; 1.0168x over previous
import jax
import jax.numpy as jnp
from jax import lax
from jax.experimental import pallas as pl
from jax.experimental.pallas import tpu as pltpu

_B, _N, _L = 16, 1024, 8
_CH = 256


_RPS = 2


def _one_row(vals, labs):
    count = jnp.zeros((_N,), jnp.float32)
    for c in range(_N // _CH):
        vi = vals[c * _CH:(c + 1) * _CH][:, None]
        li = labs[c * _CH:(c + 1) * _CH][:, None]
        eq = (vi == vals[None, :]) & (li == labs[None, :])
        count = count + jnp.sum(eq.astype(jnp.float32), axis=0)

    gids = lax.broadcasted_iota(jnp.int32, (_L, 1), 0).astype(jnp.float32)
    m = labs[None, :] == gids
    gsize = jnp.sum(jnp.where(m, 1.0, 0.0), axis=1)
    gsum = jnp.sum(jnp.where(m, vals[None, :], 0.0), axis=1)
    gmax = jnp.max(jnp.where(m, count[None, :], -1.0), axis=1)
    cand = m & (count[None, :] == gmax[:, None])
    mode = jnp.min(jnp.where(cand, vals[None, :], jnp.inf), axis=1)
    contrib = jnp.where(gsize > 0, gsum - gsize * mode, 0.0)
    return jnp.sum(contrib)


def _row_body(vals_ref, labs_ref, out_ref):
    b = pl.program_id(0)
    total = jnp.float32(0.0)
    for r in range(_RPS):
        total = total + _one_row(vals_ref[r, 0, :], labs_ref[r, 0, :])

    @pl.when(b == 0)
    def _():
        out_ref[0, 0] = jnp.float32(0.0)

    out_ref[0, 0] += total / jnp.float32(_B * _N)


def kernel(inputs, targets):
    out = pl.pallas_call(
        _row_body,
        grid=(_B // _RPS,),
        in_specs=[
            pl.BlockSpec((_RPS, 1, _N), lambda b: (b, 0, 0)),
            pl.BlockSpec((_RPS, 1, _N), lambda b: (b, 0, 0)),
        ],
        out_specs=pl.BlockSpec((1, 1), lambda b: (0, 0), memory_space=pltpu.SMEM),
        out_shape=jax.ShapeDtypeStruct((1, 1), jnp.float32),
    )(inputs.reshape(_B, 1, _N), targets.reshape(_B, 1, _N))
    return out[0, 0]


# speedup vs baseline: 2.8383x; 1.1866x over previous
import jax
import jax.numpy as jnp
from jax import lax
from jax.experimental import pallas as pl
from jax.experimental.pallas import tpu as pltpu

_B, _N, _L = 16, 1024, 8
_CH = 1024


_RPS = 2


def _one_row(vals, labs):
    count = jnp.zeros((_N,), jnp.float32)
    for c in range(_N // _CH):
        vi = vals[c * _CH:(c + 1) * _CH][:, None]
        li = labs[c * _CH:(c + 1) * _CH][:, None]
        eq = (vi == vals[None, :]) & (li == labs[None, :])
        count = count + jnp.sum(eq.astype(jnp.float32), axis=0)

    gids = lax.broadcasted_iota(jnp.int32, (_L, 1), 0).astype(jnp.float32)
    m = labs[None, :] == gids
    gsize = jnp.sum(jnp.where(m, 1.0, 0.0), axis=1)
    gsum = jnp.sum(jnp.where(m, vals[None, :], 0.0), axis=1)
    gmax = jnp.max(jnp.where(m, count[None, :], -1.0), axis=1)
    cand = m & (count[None, :] == gmax[:, None])
    mode = jnp.min(jnp.where(cand, vals[None, :], jnp.inf), axis=1)
    contrib = jnp.where(gsize > 0, gsum - gsize * mode, 0.0)
    return jnp.sum(contrib)


def _row_body(vals_ref, labs_ref, out_ref):
    b = pl.program_id(0)
    total = jnp.float32(0.0)
    for r in range(_RPS):
        total = total + _one_row(vals_ref[r, 0, :], labs_ref[r, 0, :])

    @pl.when(b == 0)
    def _():
        out_ref[0, 0] = jnp.float32(0.0)

    out_ref[0, 0] += total / jnp.float32(_B * _N)


def kernel(inputs, targets):
    out = pl.pallas_call(
        _row_body,
        grid=(_B // _RPS,),
        in_specs=[
            pl.BlockSpec((_RPS, 1, _N), lambda b: (b, 0, 0)),
            pl.BlockSpec((_RPS, 1, _N), lambda b: (b, 0, 0)),
        ],
        out_specs=pl.BlockSpec((1, 1), lambda b: (0, 0), memory_space=pltpu.SMEM),
        out_shape=jax.ShapeDtypeStruct((1, 1), jnp.float32),
    )(inputs.reshape(_B, 1, _N), targets.reshape(_B, 1, _N))
    return out[0, 0]
